# Initial kernel scaffold; baseline (speedup 1.0000x reference)
import jax
import jax.numpy as jnp
from jax.experimental import pallas as pl


def kernel(x, ln1_w, ln2_w, wq, wk, wv, wo, router_w, w1, w2):
    raise NotImplementedError("write your pallas kernel here")


# speedup vs baseline: 1.0711x; 1.0711x over previous
import functools
import math

import jax
import jax.numpy as jnp
from jax.experimental import pallas as pl

B, S, D = 1, 2048, 1024
H, DH = 16, 64
E, TOPK, FF = 8, 2, 2048
EPS = 1e-6

S_TILE = 256
NT = S // S_TILE
MOE_TILE = 512
NMT = S // MOE_TILE

_NEG = -1e30
_LN1E4 = math.log(10000.0)


def _rms(h, w):
    var = jnp.mean(h * h, axis=-1, keepdims=True)
    return h * jax.lax.rsqrt(var + EPS) * w


def _k1_body(x_ref, ln1_ref, wql_ref, wqh_ref, wkl_ref, wkh_ref, wv_ref,
             qa_ref, qb_ref, ka_ref, kb_ref, v_ref):
    i = pl.program_id(0)
    h = _rms(x_ref[...], ln1_ref[...])
    ql = jnp.dot(h, wql_ref[...], preferred_element_type=jnp.float32)
    qh = jnp.dot(h, wqh_ref[...], preferred_element_type=jnp.float32)
    kl = jnp.dot(h, wkl_ref[...], preferred_element_type=jnp.float32)
    kh = jnp.dot(h, wkh_ref[...], preferred_element_type=jnp.float32)
    v_ref[...] = jnp.dot(h, wv_ref[...], preferred_element_type=jnp.float32)
    pos = (i * S_TILE + jax.lax.broadcasted_iota(jnp.int32, (S_TILE, H * DH // 2), 0)
           ).astype(jnp.float32)
    lane = jax.lax.broadcasted_iota(jnp.int32, (S_TILE, H * DH // 2), 1) % (DH // 2)
    inv_freq = jnp.exp(lane.astype(jnp.float32) * (-2.0 * _LN1E4 / DH))
    theta = pos * inv_freq
    c = jnp.cos(theta)
    s = jnp.sin(theta)
    qa_ref[...] = ql * c - qh * s
    qb_ref[...] = qh * c + ql * s
    ka_ref[...] = kl * c - kh * s
    kb_ref[...] = kh * c + kl * s


def _k2_body(qa_ref, qb_ref, ka_ref, kb_ref, v_ref, o_ref):
    qi = pl.program_id(0)
    rowp = qi * S_TILE + jax.lax.broadcasted_iota(jnp.int32, (S_TILE, S_TILE), 0)
    scale = 1.0 / math.sqrt(DH)
    for h in range(H):
        ha = slice(h * (DH // 2), (h + 1) * (DH // 2))
        hv = slice(h * DH, (h + 1) * DH)
        q = jnp.concatenate([qa_ref[:, ha], qb_ref[:, ha]], axis=1) * scale

        def body(j, carry, q=q):
            m, l, acc = carry
            k = jnp.concatenate(
                [ka_ref[pl.ds(j * S_TILE, S_TILE), ha],
                 kb_ref[pl.ds(j * S_TILE, S_TILE), ha]], axis=1)
            v = v_ref[pl.ds(j * S_TILE, S_TILE), hv]
            s = jax.lax.dot_general(q, k, (((1,), (1,)), ((), ())),
                                    preferred_element_type=jnp.float32)
            colp = j * S_TILE + jax.lax.broadcasted_iota(
                jnp.int32, (S_TILE, S_TILE), 1)
            s = jnp.where(colp <= rowp, s, _NEG)
            m_new = jnp.maximum(m, jnp.max(s, axis=1, keepdims=True))
            p = jnp.exp(s - m_new)
            sc = jnp.exp(m - m_new)
            l_new = l * sc + jnp.sum(p, axis=1, keepdims=True)
            acc_new = acc * sc + jnp.dot(p, v, preferred_element_type=jnp.float32)
            return m_new, l_new, acc_new

        m0 = jnp.full((S_TILE, 1), _NEG, jnp.float32)
        l0 = jnp.zeros((S_TILE, 1), jnp.float32)
        a0 = jnp.zeros((S_TILE, DH), jnp.float32)
        m, l, acc = jax.lax.fori_loop(0, qi + 1, body, (m0, l0, a0))
        o_ref[:, hv] = acc / l


def _k3_body(ctx_ref, x_ref, wo_ref, ln2_ref, rw_ref, x1_ref, h2_ref, lg_ref):
    x1 = x_ref[...] + jnp.dot(ctx_ref[...], wo_ref[...],
                              preferred_element_type=jnp.float32)
    x1_ref[...] = x1
    h2 = _rms(x1, ln2_ref[...])
    h2_ref[...] = h2
    lg_ref[...] = jnp.dot(h2, rw_ref[...], preferred_element_type=jnp.float32)


def _k4_body(lg_ref, gates_ref, aux_ref):
    lg = lg_ref[...]
    mx = jnp.max(lg, axis=1, keepdims=True)
    ex = jnp.exp(lg - mx)
    probs = ex / jnp.sum(ex, axis=1, keepdims=True)
    lane = jax.lax.broadcasted_iota(jnp.int32, (S, E), 1)
    v1 = jnp.max(probs, axis=1, keepdims=True)
    i1 = jnp.min(jnp.where(probs == v1, lane, E), axis=1, keepdims=True)
    oh1 = (lane == i1).astype(jnp.float32)
    masked = jnp.where(lane == i1, _NEG, probs)
    v2 = jnp.max(masked, axis=1, keepdims=True)
    i2 = jnp.min(jnp.where(masked == v2, lane, E), axis=1, keepdims=True)
    oh2 = (lane == i2).astype(jnp.float32)
    tot = v1 + v2
    gates_ref[...] = oh1 * (v1 / tot) + oh2 * (v2 / tot)
    f = jnp.sum(oh1 + oh2, axis=0, keepdims=True) / (S * TOPK)
    pbar = jnp.sum(probs, axis=0, keepdims=True) / S
    aux_ref[...] = E * jnp.sum(f * pbar, axis=1, keepdims=True)


def _k5_body(h2_ref, x1_ref, gates_ref, w1_ref, w2_ref, out_ref):
    e = pl.program_id(1)
    lane = jax.lax.broadcasted_iota(jnp.int32, (MOE_TILE, E), 1)
    ge = jnp.sum(jnp.where(lane == e, gates_ref[...], 0.0), axis=1, keepdims=True)
    a = jnp.dot(h2_ref[...], w1_ref[0], preferred_element_type=jnp.float32)
    g = jax.nn.gelu(a)
    y = jnp.dot(g, w2_ref[0], preferred_element_type=jnp.float32)
    contrib = ge * y

    @pl.when(e == 0)
    def _():
        out_ref[...] = x1_ref[...] + contrib

    @pl.when(e != 0)
    def _():
        out_ref[...] = out_ref[...] + contrib


def kernel(x, ln1_w, ln2_w, wq, wk, wv, wo, router_w, w1, w2):
    xs = x.reshape(S, D)
    ln1 = ln1_w.reshape(1, D)
    ln2 = ln2_w.reshape(1, D)
    wq4 = wq.reshape(D, H, 2, DH // 2).transpose(0, 2, 1, 3).reshape(D, 2, D // 2)
    wk4 = wk.reshape(D, H, 2, DH // 2).transpose(0, 2, 1, 3).reshape(D, 2, D // 2)
    wql, wqh = wq4[:, 0], wq4[:, 1]
    wkl, wkh = wk4[:, 0], wk4[:, 1]

    full = lambda shape: pl.BlockSpec(shape, lambda *idx: tuple(0 for _ in shape))
    row_tile = lambda w, tile: pl.BlockSpec((tile, w), lambda i, *_: (i, 0))

    qa, qb, ka, kb, v = pl.pallas_call(
        _k1_body,
        grid=(NT,),
        in_specs=[row_tile(D, S_TILE), full((1, D)), full((D, D // 2)),
                  full((D, D // 2)), full((D, D // 2)), full((D, D // 2)),
                  full((D, D))],
        out_specs=[row_tile(D // 2, S_TILE)] * 4 + [row_tile(D, S_TILE)],
        out_shape=[jax.ShapeDtypeStruct((S, D // 2), jnp.float32)] * 4
        + [jax.ShapeDtypeStruct((S, D), jnp.float32)],
    )(xs, ln1, wql, wqh, wkl, wkh, wv)

    ctx = pl.pallas_call(
        _k2_body,
        grid=(NT,),
        in_specs=[
            row_tile(D // 2, S_TILE),
            row_tile(D // 2, S_TILE),
            full((S, D // 2)),
            full((S, D // 2)),
            full((S, D)),
        ],
        out_specs=row_tile(D, S_TILE),
        out_shape=jax.ShapeDtypeStruct((S, D), jnp.float32),
    )(qa, qb, ka, kb, v)

    x1, h2, logits = pl.pallas_call(
        _k3_body,
        grid=(NT,),
        in_specs=[row_tile(D, S_TILE), row_tile(D, S_TILE), full((D, D)),
                  full((1, D)), full((D, E))],
        out_specs=[row_tile(D, S_TILE), row_tile(D, S_TILE), row_tile(E, S_TILE)],
        out_shape=[jax.ShapeDtypeStruct((S, D), jnp.float32),
                   jax.ShapeDtypeStruct((S, D), jnp.float32),
                   jax.ShapeDtypeStruct((S, E), jnp.float32)],
    )(ctx, xs, wo, ln2, router_w)

    gates, aux = pl.pallas_call(
        _k4_body,
        grid=(1,),
        in_specs=[full((S, E))],
        out_specs=[full((S, E)), full((1, 1))],
        out_shape=[jax.ShapeDtypeStruct((S, E), jnp.float32),
                   jax.ShapeDtypeStruct((1, 1), jnp.float32)],
    )(logits)

    x2 = pl.pallas_call(
        _k5_body,
        grid=(NMT, E),
        in_specs=[
            pl.BlockSpec((MOE_TILE, D), lambda t, e: (t, 0)),
            pl.BlockSpec((MOE_TILE, D), lambda t, e: (t, 0)),
            pl.BlockSpec((MOE_TILE, E), lambda t, e: (t, 0)),
            pl.BlockSpec((1, D, FF), lambda t, e: (e, 0, 0)),
            pl.BlockSpec((1, FF, D), lambda t, e: (e, 0, 0)),
        ],
        out_specs=pl.BlockSpec((MOE_TILE, D), lambda t, e: (t, 0)),
        out_shape=jax.ShapeDtypeStruct((S, D), jnp.float32),
    )(h2, x1, gates, w1, w2)

    return x2.reshape(B, S, D), aux.reshape(())
